# baseline (speedup 1.0000x reference)
import functools

import jax
import jax.numpy as jnp
from jax import lax
from jax.experimental import pallas as pl
from jax.experimental.pallas import tpu as pltpu
from jax.experimental.pallas import tpu_sc as plsc

_NC = 2
_NS = 16
_NW = _NC * _NS
_L = 16

_EMB = 64
_NEG = 20
_DC = _EMB // _L

_GDN = lax.GatherDimensionNumbers(
    offset_dims=(), collapsed_slice_dims=(0,), start_index_map=(0,))


def _hsum_all_lanes(v, lanes):
    for s in (1, 2, 4, 8):
        perm = jnp.reshape(lanes ^ s, (_L, 1))
        p = lax.gather(v, perm, _GDN, slice_sizes=(1,),
                       mode=lax.GatherScatterMode.PROMISE_IN_BOUNDS)
        v = v + p
    return v


def _prep_table(u_emb, v_emb):
    V = u_emb.shape[0]
    R = 7936
    grid = pl.cdiv(V, R)
    R2 = R // 2

    def body(u_ref, v_ref, o_ref):
        t = jnp.concatenate([u_ref[...], v_ref[...]], axis=0).T
        a = jnp.concatenate([t[:, 0:32], t[:, 64:96]], axis=1)
        b = jnp.concatenate([t[:, 32:64], t[:, 96:128]], axis=1)
        au = lax.bitcast_convert_type(a.astype(jnp.bfloat16),
                                      jnp.uint16).astype(jnp.uint32)
        bu = lax.bitcast_convert_type(b.astype(jnp.bfloat16),
                                      jnp.uint16).astype(jnp.uint32)
        x = au | (bu << 16)
        o_ref[...] = jnp.concatenate([x[:R2], x[R2:]], axis=1)

    return pl.pallas_call(
        body,
        grid=(grid,),
        in_specs=[
            pl.BlockSpec((_EMB, R), lambda i: (0, i)),
            pl.BlockSpec((_EMB, R), lambda i: (0, i)),
        ],
        out_specs=pl.BlockSpec((R2, 2 * _EMB), lambda i: (i, 0)),
        out_shape=jax.ShapeDtypeStruct((grid * R2, 2 * _EMB), jnp.uint32),
    )(u_emb.T, v_emb.T)


def _sc_scores(u_pos, v_pos, v_neg, w2_table):
    B = u_pos.shape[0]
    bpw = B // _NW
    CB = 16
    n_chunks = bpw // CB
    gpc = (CB * _NEG) // 64
    neg_gj = n_chunks * gpc

    u_pos3 = u_pos.reshape(_NW, n_chunks, CB)
    v_pos3 = v_pos.reshape(_NW, n_chunks, CB)
    v_neg3 = v_neg.reshape(_NW, neg_gj, 64)

    mesh = plsc.VectorSubcoreMesh(core_axis_name="c", subcore_axis_name="s")

    @functools.partial(
        pl.kernel,
        mesh=mesh,
        compiler_params=pltpu.CompilerParams(use_tc_tiling_on_sc=False,
                                             needs_layout_passes=False),
        out_type=[
            jax.ShapeDtypeStruct((B,), jnp.float32),
            jax.ShapeDtypeStruct((B,), jnp.float32),
        ],
        scratch_types=[
            pltpu.VMEM((n_chunks, CB), jnp.int32),
            pltpu.VMEM((n_chunks, CB), jnp.int32),
            pltpu.VMEM((neg_gj, 64), jnp.int32),
            pltpu.VMEM((2, CB, _EMB // 2), jnp.uint32),
            pltpu.VMEM((2, CB, _EMB // 2), jnp.uint32),
            pltpu.VMEM((2, CB * _NEG, _EMB // 2), jnp.uint32),
            pltpu.VMEM((bpw,), jnp.float32),
            pltpu.VMEM((bpw,), jnp.float32),
            pltpu.SemaphoreType.DMA,
            pltpu.SemaphoreType.DMA,
        ],
    )
    def sc_kernel(u_pos_hbm, v_pos_hbm, v_neg_hbm, w_hbm,
                  score_hbm, negscore_hbm,
                  uidx_v, vidx_v, nidx_v, u_buf2, v_buf2, neg_buf2,
                  score_v, negs_v, sem_a, sem_b):
        wid = lax.axis_index("s") * _NC + lax.axis_index("c")
        base = wid * bpw

        pltpu.sync_copy(u_pos_hbm.at[wid], uidx_v)
        pltpu.sync_copy(v_pos_hbm.at[wid], vidx_v)
        pltpu.sync_copy(v_neg_hbm.at[wid], nidx_v)

        lanes = lax.iota(jnp.int32, _L)
        zeros = jnp.zeros((_L,), jnp.float32)

        def copies(c, p, sem):
            yield w_hbm.at[uidx_v.at[c]], u_buf2.at[p], sem
            yield w_hbm.at[vidx_v.at[c]], v_buf2.at[p], sem
            for j in range(gpc):
                yield (w_hbm.at[nidx_v.at[c * gpc + j]],
                       neg_buf2.at[p, pl.ds(j * 64, 64)], sem)

        def fire(c, p, sem):
            for src, dst, s in copies(c, p, sem):
                pltpu.async_copy(src, dst, s)

        def drain(c, p, sem):
            for src, dst, s in copies(c, p, sem):
                pltpu.make_async_copy(src, dst, s).wait()

        def compute(c, p):
            u_buf = u_buf2.at[p]
            v_buf = v_buf2.at[p]
            neg_buf = neg_buf2.at[p]

            def unpk(w):
                a = plsc.bitcast(w << 16, jnp.float32)
                b = plsc.bitcast(w & jnp.uint32(0xFFFF0000), jnp.float32)
                return a, b

            def j_body(j, carry):
                svec, nvec = carry
                r = j * _NEG
                t = zeros
                tn = zeros
                for h in range(2):
                    sl = pl.ds(h * _L, _L)
                    ua, ub = unpk(u_buf[j, sl])
                    va, vb = unpk(v_buf[j, sl])
                    t = t + ua * va + ub * vb
                    na, nb = unpk(neg_buf[r, sl])
                    for k in range(1, _NEG):
                        xa, xb = unpk(neg_buf[r + k, sl])
                        na = na + xa
                        nb = nb + xb
                    tn = tn + na * ua + nb * ub
                svec = jnp.where(lanes == j, _hsum_all_lanes(t, lanes), svec)
                nvec = jnp.where(lanes == j, _hsum_all_lanes(tn, lanes),
                                 nvec)
                return svec, nvec

            svec, nvec = lax.fori_loop(0, _L, j_body, (zeros, zeros))
            score_v[pl.ds(c * CB, CB)] = svec
            negs_v[pl.ds(c * CB, CB)] = nvec

        fire(0, 0, sem_a)

        def pair_body(c2, _):
            c0 = 2 * c2
            c1 = c0 + 1
            fire(c1, 1, sem_b)
            drain(c0, 0, sem_a)
            compute(c0, 0)

            @pl.when(c2 + 1 < n_chunks // 2)
            def _():
                fire(c0 + 2, 0, sem_a)

            drain(c1, 1, sem_b)
            compute(c1, 1)
            return 0

        lax.fori_loop(0, n_chunks // 2, pair_body, 0)

        pltpu.sync_copy(score_v, score_hbm.at[pl.ds(base, bpw)])
        pltpu.sync_copy(negs_v, negscore_hbm.at[pl.ds(base, bpw)])

    return sc_kernel(u_pos3, v_pos3, v_neg3, w2_table)


def _tc_loss(score, neg_score):
    B = score.shape[0]
    s2 = score.reshape(128, B // 128)
    n2 = neg_score.reshape(128, B // 128)

    def body(s_ref, n_ref, o_ref):
        s = s_ref[...]
        n = n_ref[...]
        ls = jnp.minimum(s, 0.0) - jnp.log1p(jnp.exp(-jnp.abs(s)))
        ln = jnp.minimum(-n, 0.0) - jnp.log1p(jnp.exp(-jnp.abs(n)))
        o_ref[...] = jnp.reshape((jnp.sum(ls) + jnp.sum(ln)) * (-1.0 / B),
                                 (1, 1))

    out = pl.pallas_call(
        body,
        out_shape=jax.ShapeDtypeStruct((1, 1), jnp.float32),
    )(s2, n2)
    return out.reshape(())


def kernel(u_pos, v_pos, v_neg, visual_pos, batch_size,
           u_emb, v_emb, visual_table, gate_W, gate_b):
    w32 = _prep_table(u_emb, v_emb)
    w2 = w32.reshape(4 * w32.shape[0], _EMB // 2)
    R, R2 = 7936, 3968

    def urow(g):
        i, p = jnp.divmod(g, R)
        h = (p >= R2).astype(jnp.int32)
        return 4 * (i * R2 + p - h * R2) + 2 * h
    score, neg_score = _sc_scores(urow(u_pos), urow(v_pos) + 1,
                                  urow(v_neg) + 1, w2)
    return _tc_loss(score, neg_score)


# speedup vs baseline: 1.2998x; 1.2998x over previous
import functools

import jax
import jax.numpy as jnp
from jax import lax
from jax.experimental import pallas as pl
from jax.experimental.pallas import tpu as pltpu
from jax.experimental.pallas import tpu_sc as plsc

_NC = 2
_NS = 16
_NW = _NC * _NS
_L = 16

_EMB = 64
_NEG = 20
_DC = _EMB // _L

_GDN = lax.GatherDimensionNumbers(
    offset_dims=(), collapsed_slice_dims=(0,), start_index_map=(0,))


def _hsum_all_lanes(v, lanes):
    for s in (1, 2, 4, 8):
        perm = jnp.reshape(lanes ^ s, (_L, 1))
        p = lax.gather(v, perm, _GDN, slice_sizes=(1,),
                       mode=lax.GatherScatterMode.PROMISE_IN_BOUNDS)
        v = v + p
    return v


def _prep_table(u_emb, v_emb):
    V = u_emb.shape[0]
    R = 7936
    grid = pl.cdiv(V, R)
    R2 = R // 2

    def body(u_ref, v_ref, o_ref):
        au = lax.bitcast_convert_type(u_ref[...].astype(jnp.bfloat16),
                                      jnp.uint16).astype(jnp.uint32)
        bu = lax.bitcast_convert_type(v_ref[...].astype(jnp.bfloat16),
                                      jnp.uint16).astype(jnp.uint32)
        x = au | (bu << 16)
        stacked = jnp.concatenate([x[:, :R2], x[:, R2:]], axis=0)
        o_ref[...] = stacked.T

    return pl.pallas_call(
        body,
        grid=(grid,),
        in_specs=[
            pl.BlockSpec((_EMB, R), lambda i: (0, i)),
            pl.BlockSpec((_EMB, R), lambda i: (0, i)),
        ],
        out_specs=pl.BlockSpec((R2, 2 * _EMB), lambda i: (i, 0)),
        out_shape=jax.ShapeDtypeStruct((grid * R2, 2 * _EMB), jnp.uint32),
    )(u_emb.T, v_emb.T)


def _sc_scores(u_pos, v_pos, v_neg, w2_table):
    B = u_pos.shape[0]
    bpw = B // _NW
    CB = 16
    n_chunks = bpw // CB
    gpc = (CB * _NEG) // 64
    neg_gj = n_chunks * gpc

    u_pos3 = u_pos.reshape(_NW, n_chunks, CB)
    v_pos3 = v_pos.reshape(_NW, n_chunks, CB)
    v_neg3 = v_neg.reshape(_NW, neg_gj, 64)

    mesh = plsc.VectorSubcoreMesh(core_axis_name="c", subcore_axis_name="s")

    @functools.partial(
        pl.kernel,
        mesh=mesh,
        compiler_params=pltpu.CompilerParams(use_tc_tiling_on_sc=False,
                                             needs_layout_passes=False),
        out_type=[
            jax.ShapeDtypeStruct((B,), jnp.float32),
            jax.ShapeDtypeStruct((B,), jnp.float32),
        ],
        scratch_types=[
            pltpu.VMEM((n_chunks, CB), jnp.int32),
            pltpu.VMEM((n_chunks, CB), jnp.int32),
            pltpu.VMEM((neg_gj, 64), jnp.int32),
            pltpu.VMEM((2, CB, _EMB), jnp.uint32),
            pltpu.VMEM((2, CB, _EMB), jnp.uint32),
            pltpu.VMEM((2, CB * _NEG, _EMB), jnp.uint32),
            pltpu.VMEM((bpw,), jnp.float32),
            pltpu.VMEM((bpw,), jnp.float32),
            pltpu.SemaphoreType.DMA,
            pltpu.SemaphoreType.DMA,
        ],
    )
    def sc_kernel(u_pos_hbm, v_pos_hbm, v_neg_hbm, w_hbm,
                  score_hbm, negscore_hbm,
                  uidx_v, vidx_v, nidx_v, u_buf2, v_buf2, neg_buf2,
                  score_v, negs_v, sem_a, sem_b):
        wid = lax.axis_index("s") * _NC + lax.axis_index("c")
        base = wid * bpw

        pltpu.sync_copy(u_pos_hbm.at[wid], uidx_v)
        pltpu.sync_copy(v_pos_hbm.at[wid], vidx_v)
        pltpu.sync_copy(v_neg_hbm.at[wid], nidx_v)

        lanes = lax.iota(jnp.int32, _L)
        zeros = jnp.zeros((_L,), jnp.float32)

        def copies(c, p, sem):
            yield w_hbm.at[uidx_v.at[c]], u_buf2.at[p], sem
            yield w_hbm.at[vidx_v.at[c]], v_buf2.at[p], sem
            for j in range(gpc):
                yield (w_hbm.at[nidx_v.at[c * gpc + j]],
                       neg_buf2.at[p, pl.ds(j * 64, 64)], sem)

        def fire(c, p, sem):
            for src, dst, s in copies(c, p, sem):
                pltpu.async_copy(src, dst, s)

        def drain(c, p, sem):
            for src, dst, s in copies(c, p, sem):
                pltpu.make_async_copy(src, dst, s).wait()

        def compute(c, p):
            u_buf = u_buf2.at[p]
            v_buf = v_buf2.at[p]
            neg_buf = neg_buf2.at[p]

            def u_part(w):
                return plsc.bitcast(w << 16, jnp.float32)

            def v_part(w):
                return plsc.bitcast(w & jnp.uint32(0xFFFF0000), jnp.float32)

            def j_body(j, carry):
                svec, nvec = carry
                r = j * _NEG
                t = zeros
                tn = zeros
                for dc in range(_DC):
                    sl = pl.ds(dc * _L, _L)
                    ud = u_part(u_buf[j, sl])
                    t = t + ud * v_part(v_buf[j, sl])
                    a = v_part(neg_buf[r, sl])
                    for k in range(1, _NEG):
                        a = a + v_part(neg_buf[r + k, sl])
                    tn = tn + a * ud
                svec = jnp.where(lanes == j, _hsum_all_lanes(t, lanes), svec)
                nvec = jnp.where(lanes == j, _hsum_all_lanes(tn, lanes),
                                 nvec)
                return svec, nvec

            svec, nvec = lax.fori_loop(0, _L, j_body, (zeros, zeros))
            score_v[pl.ds(c * CB, CB)] = svec
            negs_v[pl.ds(c * CB, CB)] = nvec

        fire(0, 0, sem_a)

        def pair_body(c2, _):
            c0 = 2 * c2
            c1 = c0 + 1
            fire(c1, 1, sem_b)
            drain(c0, 0, sem_a)
            compute(c0, 0)

            @pl.when(c2 + 1 < n_chunks // 2)
            def _():
                fire(c0 + 2, 0, sem_a)

            drain(c1, 1, sem_b)
            compute(c1, 1)
            return 0

        lax.fori_loop(0, n_chunks // 2, pair_body, 0)

        pltpu.sync_copy(score_v, score_hbm.at[pl.ds(base, bpw)])
        pltpu.sync_copy(negs_v, negscore_hbm.at[pl.ds(base, bpw)])

    return sc_kernel(u_pos3, v_pos3, v_neg3, w2_table)


def _tc_loss(score, neg_score):
    B = score.shape[0]
    s2 = score.reshape(128, B // 128)
    n2 = neg_score.reshape(128, B // 128)

    def body(s_ref, n_ref, o_ref):
        s = s_ref[...]
        n = n_ref[...]
        ls = jnp.minimum(s, 0.0) - jnp.log1p(jnp.exp(-jnp.abs(s)))
        ln = jnp.minimum(-n, 0.0) - jnp.log1p(jnp.exp(-jnp.abs(n)))
        o_ref[...] = jnp.reshape((jnp.sum(ls) + jnp.sum(ln)) * (-1.0 / B),
                                 (1, 1))

    out = pl.pallas_call(
        body,
        out_shape=jax.ShapeDtypeStruct((1, 1), jnp.float32),
    )(s2, n2)
    return out.reshape(())


def kernel(u_pos, v_pos, v_neg, visual_pos, batch_size,
           u_emb, v_emb, visual_table, gate_W, gate_b):
    w32 = _prep_table(u_emb, v_emb)
    w2 = w32.reshape(2 * w32.shape[0], _EMB)
    R, R2 = 7936, 3968

    def vrow(g):
        i, p = jnp.divmod(g, R)
        h = (p >= R2).astype(jnp.int32)
        return 2 * (i * R2 + p - h * R2) + h

    score, neg_score = _sc_scores(vrow(u_pos), vrow(v_pos), vrow(v_neg),
                                  w2)
    return _tc_loss(score, neg_score)


# speedup vs baseline: 1.3347x; 1.0268x over previous
import functools

import jax
import jax.numpy as jnp
from jax import lax
from jax.experimental import pallas as pl
from jax.experimental.pallas import tpu as pltpu
from jax.experimental.pallas import tpu_sc as plsc

_NC = 2
_NS = 16
_NW = _NC * _NS
_L = 16

_EMB = 64
_NEG = 20
_DC = _EMB // _L

_GDN = lax.GatherDimensionNumbers(
    offset_dims=(), collapsed_slice_dims=(0,), start_index_map=(0,))


def _hsum_all_lanes(v, lanes):
    for s in (1, 2, 4, 8):
        perm = jnp.reshape(lanes ^ s, (_L, 1))
        p = lax.gather(v, perm, _GDN, slice_sizes=(1,),
                       mode=lax.GatherScatterMode.PROMISE_IN_BOUNDS)
        v = v + p
    return v


def _prep_table(u_emb, v_emb):
    V = u_emb.shape[0]
    R = 8192
    grid = pl.cdiv(V, R)
    R4 = R // 4

    def body(u_ref, v_ref, ou_ref, ov_ref):
        def packed(ref):
            bits = lax.bitcast_convert_type(
                ref[...].astype(jnp.bfloat16), jnp.uint16).astype(jnp.uint32)
            x = bits[:32] | (bits[32:] << 16)
            stacked = jnp.concatenate(
                [x[:, i * R4:(i + 1) * R4] for i in range(4)], axis=0)
            return stacked.T

        ou_ref[...] = packed(u_ref)
        ov_ref[...] = packed(v_ref)

    return pl.pallas_call(
        body,
        grid=(grid,),
        in_specs=[
            pl.BlockSpec((_EMB, R), lambda i: (0, i)),
            pl.BlockSpec((_EMB, R), lambda i: (0, i)),
        ],
        out_specs=[
            pl.BlockSpec((R4, 2 * _EMB), lambda i: (i, 0)),
            pl.BlockSpec((R4, 2 * _EMB), lambda i: (i, 0)),
        ],
        out_shape=[
            jax.ShapeDtypeStruct((grid * R4, 2 * _EMB), jnp.uint32),
            jax.ShapeDtypeStruct((grid * R4, 2 * _EMB), jnp.uint32),
        ],
    )(u_emb.T, v_emb.T)


def _sc_scores(u_pos, v_pos, v_neg, wu_table, wv_table):
    B = u_pos.shape[0]
    bpw = B // _NW
    CB = 16
    n_chunks = bpw // CB
    gpc = (CB * _NEG) // 64
    neg_gj = n_chunks * gpc

    u_pos3 = u_pos.reshape(_NW, n_chunks, CB)
    v_pos3 = v_pos.reshape(_NW, n_chunks, CB)
    v_neg3 = v_neg.reshape(_NW, neg_gj, 64)

    mesh = plsc.VectorSubcoreMesh(core_axis_name="c", subcore_axis_name="s")

    @functools.partial(
        pl.kernel,
        mesh=mesh,
        compiler_params=pltpu.CompilerParams(use_tc_tiling_on_sc=False,
                                             needs_layout_passes=False),
        out_type=[
            jax.ShapeDtypeStruct((B,), jnp.float32),
            jax.ShapeDtypeStruct((B,), jnp.float32),
        ],
        scratch_types=[
            pltpu.VMEM((n_chunks, CB), jnp.int32),
            pltpu.VMEM((n_chunks, CB), jnp.int32),
            pltpu.VMEM((neg_gj, 64), jnp.int32),
            pltpu.VMEM((2, CB, _EMB // 2), jnp.uint32),
            pltpu.VMEM((2, CB, _EMB // 2), jnp.uint32),
            pltpu.VMEM((2, CB * _NEG, _EMB // 2), jnp.uint32),
            pltpu.VMEM((bpw,), jnp.float32),
            pltpu.VMEM((bpw,), jnp.float32),
            pltpu.SemaphoreType.DMA,
            pltpu.SemaphoreType.DMA,
        ],
    )
    def sc_kernel(u_pos_hbm, v_pos_hbm, v_neg_hbm, wu_hbm, wv_hbm,
                  score_hbm, negscore_hbm,
                  uidx_v, vidx_v, nidx_v, u_buf2, v_buf2, neg_buf2,
                  score_v, negs_v, sem_a, sem_b):
        wid = lax.axis_index("s") * _NC + lax.axis_index("c")
        base = wid * bpw

        pltpu.sync_copy(u_pos_hbm.at[wid], uidx_v)
        pltpu.sync_copy(v_pos_hbm.at[wid], vidx_v)
        pltpu.sync_copy(v_neg_hbm.at[wid], nidx_v)

        lanes = lax.iota(jnp.int32, _L)
        zeros = jnp.zeros((_L,), jnp.float32)

        def copies(c, p, sem):
            yield wu_hbm.at[uidx_v.at[c]], u_buf2.at[p], sem
            yield wv_hbm.at[vidx_v.at[c]], v_buf2.at[p], sem
            for j in range(gpc):
                yield (wv_hbm.at[nidx_v.at[c * gpc + j]],
                       neg_buf2.at[p, pl.ds(j * 64, 64)], sem)

        def fire(c, p, sem):
            for src, dst, s in copies(c, p, sem):
                pltpu.async_copy(src, dst, s)

        def drain(c, p, sem):
            for src, dst, s in copies(c, p, sem):
                pltpu.make_async_copy(src, dst, s).wait()

        def compute(c, p):
            u_buf = u_buf2.at[p]
            v_buf = v_buf2.at[p]
            neg_buf = neg_buf2.at[p]

            def unpk(w):
                a = plsc.bitcast(w << 16, jnp.float32)
                b = plsc.bitcast(w & jnp.uint32(0xFFFF0000), jnp.float32)
                return a, b

            def j_body(j, carry):
                svec, nvec = carry
                r = j * _NEG
                t = zeros
                tn = zeros
                for h in range(2):
                    sl = pl.ds(h * _L, _L)
                    ua, ub = unpk(u_buf[j, sl])
                    va, vb = unpk(v_buf[j, sl])
                    t = t + ua * va + ub * vb
                    na, nb = unpk(neg_buf[r, sl])
                    for k in range(1, _NEG):
                        xa, xb = unpk(neg_buf[r + k, sl])
                        na = na + xa
                        nb = nb + xb
                    tn = tn + na * ua + nb * ub
                svec = jnp.where(lanes == j, _hsum_all_lanes(t, lanes), svec)
                nvec = jnp.where(lanes == j, _hsum_all_lanes(tn, lanes),
                                 nvec)
                return svec, nvec

            svec, nvec = lax.fori_loop(0, _L, j_body, (zeros, zeros))
            score_v[pl.ds(c * CB, CB)] = svec
            negs_v[pl.ds(c * CB, CB)] = nvec

        fire(0, 0, sem_a)

        def pair_body(c2, _):
            c0 = 2 * c2
            c1 = c0 + 1
            fire(c1, 1, sem_b)
            drain(c0, 0, sem_a)
            compute(c0, 0)

            @pl.when(c2 + 1 < n_chunks // 2)
            def _():
                fire(c0 + 2, 0, sem_a)

            drain(c1, 1, sem_b)
            compute(c1, 1)
            return 0

        lax.fori_loop(0, n_chunks // 2, pair_body, 0)

        pltpu.sync_copy(score_v, score_hbm.at[pl.ds(base, bpw)])
        pltpu.sync_copy(negs_v, negscore_hbm.at[pl.ds(base, bpw)])

    return sc_kernel(u_pos3, v_pos3, v_neg3, wu_table, wv_table)


def _tc_loss(score, neg_score):
    B = score.shape[0]
    s2 = score.reshape(128, B // 128)
    n2 = neg_score.reshape(128, B // 128)

    def body(s_ref, n_ref, o_ref):
        s = s_ref[...]
        n = n_ref[...]
        ls = jnp.minimum(s, 0.0) - jnp.log1p(jnp.exp(-jnp.abs(s)))
        ln = jnp.minimum(-n, 0.0) - jnp.log1p(jnp.exp(-jnp.abs(n)))
        o_ref[...] = jnp.reshape((jnp.sum(ls) + jnp.sum(ln)) * (-1.0 / B),
                                 (1, 1))

    out = pl.pallas_call(
        body,
        out_shape=jax.ShapeDtypeStruct((1, 1), jnp.float32),
    )(s2, n2)
    return out.reshape(())


def kernel(u_pos, v_pos, v_neg, visual_pos, batch_size,
           u_emb, v_emb, visual_table, gate_W, gate_b):
    wu32, wv32 = _prep_table(u_emb, v_emb)
    wu = wu32.reshape(4 * wu32.shape[0], _EMB // 2)
    wv = wv32.reshape(4 * wv32.shape[0], _EMB // 2)
    R = 8192
    R4 = R // 4

    def vrow(g):
        i, p = jnp.divmod(g, R)
        h, q = jnp.divmod(p, R4)
        return 4 * (i * R4 + q) + h

    score, neg_score = _sc_scores(vrow(u_pos), vrow(v_pos), vrow(v_neg),
                                  wu, wv)
    return _tc_loss(score, neg_score)


# speedup vs baseline: 1.3869x; 1.0391x over previous
import functools

import jax
import jax.numpy as jnp
from jax import lax
from jax.experimental import pallas as pl
from jax.experimental.pallas import tpu as pltpu
from jax.experimental.pallas import tpu_sc as plsc

_NC = 2
_NS = 16
_NW = _NC * _NS
_L = 16

_EMB = 64
_NEG = 20
_DC = _EMB // _L

_GDN = lax.GatherDimensionNumbers(
    offset_dims=(), collapsed_slice_dims=(0,), start_index_map=(0,))


def _hsum_all_lanes(v, lanes):
    for s in (1, 2, 4, 8):
        perm = jnp.reshape(lanes ^ s, (_L, 1))
        p = lax.gather(v, perm, _GDN, slice_sizes=(1,),
                       mode=lax.GatherScatterMode.PROMISE_IN_BOUNDS)
        v = v + p
    return v


def _prep_table(u_emb, v_emb):
    V = u_emb.shape[0]
    R = 16384
    grid = pl.cdiv(V, R)
    R4 = R // 4

    def body(u_ref, v_ref, ou_ref, ov_ref):
        def packed(ref):
            bits = lax.bitcast_convert_type(
                ref[...].astype(jnp.bfloat16), jnp.uint16).astype(jnp.uint32)
            x = bits[:32] | (bits[32:] << 16)
            stacked = jnp.concatenate(
                [x[:, i * R4:(i + 1) * R4] for i in range(4)], axis=0)
            return stacked.T

        ou_ref[...] = packed(u_ref)
        ov_ref[...] = packed(v_ref)

    return pl.pallas_call(
        body,
        grid=(grid,),
        in_specs=[
            pl.BlockSpec((_EMB, R), lambda i: (0, i)),
            pl.BlockSpec((_EMB, R), lambda i: (0, i)),
        ],
        out_specs=[
            pl.BlockSpec((R4, 2 * _EMB), lambda i: (i, 0)),
            pl.BlockSpec((R4, 2 * _EMB), lambda i: (i, 0)),
        ],
        out_shape=[
            jax.ShapeDtypeStruct((grid * R4, 2 * _EMB), jnp.uint32),
            jax.ShapeDtypeStruct((grid * R4, 2 * _EMB), jnp.uint32),
        ],
    )(u_emb.T, v_emb.T)


def _sc_scores(u_pos, v_pos, v_neg, wu_table, wv_table):
    B = u_pos.shape[0]
    bpw = B // _NW
    CB = 16
    n_chunks = bpw // CB
    gpc = (CB * _NEG) // 64
    neg_gj = n_chunks * gpc

    u_pos3 = u_pos.reshape(_NW, n_chunks, CB)
    v_pos3 = v_pos.reshape(_NW, n_chunks, CB)
    v_neg3 = v_neg.reshape(_NW, neg_gj, 64)

    mesh = plsc.VectorSubcoreMesh(core_axis_name="c", subcore_axis_name="s")

    @functools.partial(
        pl.kernel,
        mesh=mesh,
        compiler_params=pltpu.CompilerParams(use_tc_tiling_on_sc=False,
                                             needs_layout_passes=False),
        out_type=[
            jax.ShapeDtypeStruct((B,), jnp.float32),
            jax.ShapeDtypeStruct((B,), jnp.float32),
        ],
        scratch_types=[
            pltpu.VMEM((n_chunks, CB), jnp.int32),
            pltpu.VMEM((n_chunks, CB), jnp.int32),
            pltpu.VMEM((neg_gj, 64), jnp.int32),
            pltpu.VMEM((2, CB, _EMB // 2), jnp.uint32),
            pltpu.VMEM((2, CB, _EMB // 2), jnp.uint32),
            pltpu.VMEM((2, CB * _NEG, _EMB // 2), jnp.uint32),
            pltpu.VMEM((bpw,), jnp.float32),
            pltpu.VMEM((bpw,), jnp.float32),
            pltpu.SemaphoreType.DMA,
            pltpu.SemaphoreType.DMA,
        ],
    )
    def sc_kernel(u_pos_hbm, v_pos_hbm, v_neg_hbm, wu_hbm, wv_hbm,
                  score_hbm, negscore_hbm,
                  uidx_v, vidx_v, nidx_v, u_buf2, v_buf2, neg_buf2,
                  score_v, negs_v, sem_a, sem_b):
        wid = lax.axis_index("s") * _NC + lax.axis_index("c")
        base = wid * bpw

        pltpu.sync_copy(u_pos_hbm.at[wid], uidx_v)
        pltpu.sync_copy(v_pos_hbm.at[wid], vidx_v)
        pltpu.sync_copy(v_neg_hbm.at[wid], nidx_v)

        lanes = lax.iota(jnp.int32, _L)
        zeros = jnp.zeros((_L,), jnp.float32)

        def copies(c, p, sem):
            yield wu_hbm.at[uidx_v.at[c]], u_buf2.at[p], sem
            yield wv_hbm.at[vidx_v.at[c]], v_buf2.at[p], sem
            for j in range(gpc):
                yield (wv_hbm.at[nidx_v.at[c * gpc + j]],
                       neg_buf2.at[p, pl.ds(j * 64, 64)], sem)

        def fire(c, p, sem):
            for src, dst, s in copies(c, p, sem):
                pltpu.async_copy(src, dst, s)

        def drain(c, p, sem):
            for src, dst, s in copies(c, p, sem):
                pltpu.make_async_copy(src, dst, s).wait()

        def compute(c, p):
            u_buf = u_buf2.at[p]
            v_buf = v_buf2.at[p]
            neg_buf = neg_buf2.at[p]

            def unpk(w):
                a = plsc.bitcast(w << 16, jnp.float32)
                b = plsc.bitcast(w & jnp.uint32(0xFFFF0000), jnp.float32)
                return a, b

            def j_body(j, carry):
                svec, nvec = carry
                r = j * _NEG
                t = zeros
                tn = zeros
                for h in range(2):
                    sl = pl.ds(h * _L, _L)
                    ua, ub = unpk(u_buf[j, sl])
                    va, vb = unpk(v_buf[j, sl])
                    t = t + ua * va + ub * vb
                    na, nb = unpk(neg_buf[r, sl])
                    for k in range(1, _NEG):
                        xa, xb = unpk(neg_buf[r + k, sl])
                        na = na + xa
                        nb = nb + xb
                    tn = tn + na * ua + nb * ub
                svec = jnp.where(lanes == j, _hsum_all_lanes(t, lanes), svec)
                nvec = jnp.where(lanes == j, _hsum_all_lanes(tn, lanes),
                                 nvec)
                return svec, nvec

            svec, nvec = lax.fori_loop(0, _L, j_body, (zeros, zeros))
            score_v[pl.ds(c * CB, CB)] = svec
            negs_v[pl.ds(c * CB, CB)] = nvec

        fire(0, 0, sem_a)

        def pair_body(c2, _):
            c0 = 2 * c2
            c1 = c0 + 1
            fire(c1, 1, sem_b)
            drain(c0, 0, sem_a)
            compute(c0, 0)

            @pl.when(c2 + 1 < n_chunks // 2)
            def _():
                fire(c0 + 2, 0, sem_a)

            drain(c1, 1, sem_b)
            compute(c1, 1)
            return 0

        lax.fori_loop(0, n_chunks // 2, pair_body, 0)

        pltpu.sync_copy(score_v, score_hbm.at[pl.ds(base, bpw)])
        pltpu.sync_copy(negs_v, negscore_hbm.at[pl.ds(base, bpw)])

    return sc_kernel(u_pos3, v_pos3, v_neg3, wu_table, wv_table)


def _tc_loss(score, neg_score):
    B = score.shape[0]
    s2 = score.reshape(128, B // 128)
    n2 = neg_score.reshape(128, B // 128)

    def body(s_ref, n_ref, o_ref):
        s = s_ref[...]
        n = n_ref[...]
        ls = jnp.minimum(s, 0.0) - jnp.log1p(jnp.exp(-jnp.abs(s)))
        ln = jnp.minimum(-n, 0.0) - jnp.log1p(jnp.exp(-jnp.abs(n)))
        o_ref[...] = jnp.reshape((jnp.sum(ls) + jnp.sum(ln)) * (-1.0 / B),
                                 (1, 1))

    out = pl.pallas_call(
        body,
        out_shape=jax.ShapeDtypeStruct((1, 1), jnp.float32),
    )(s2, n2)
    return out.reshape(())


def kernel(u_pos, v_pos, v_neg, visual_pos, batch_size,
           u_emb, v_emb, visual_table, gate_W, gate_b):
    wu32, wv32 = _prep_table(u_emb, v_emb)
    wu = wu32.reshape(4 * wu32.shape[0], _EMB // 2)
    wv = wv32.reshape(4 * wv32.shape[0], _EMB // 2)
    R = 16384
    R4 = R // 4

    def vrow(g):
        i, p = jnp.divmod(g, R)
        h, q = jnp.divmod(p, R4)
        return 4 * (i * R4 + q) + h

    score, neg_score = _sc_scores(vrow(u_pos), vrow(v_pos), vrow(v_neg),
                                  wu, wv)
    return _tc_loss(score, neg_score)
